# baseline (speedup 1.0000x reference)
import jax
import jax.numpy as jnp
from jax import lax
from jax.experimental import pallas as pl
from jax.experimental.pallas import tpu as pltpu
from jax.experimental.pallas import tpu_sc as plsc

_LB = 1024
_B = 16


def _routing_body(mask_hbm, steps_hbm, steps_out_hbm, maskp_hbm, perm_hbm,
                  src_hbm, mask_v, steps_v, steps_o_v, maskp_v, perm_v,
                  src_v, unm_v):
    wid = lax.axis_index("s") * 2 + lax.axis_index("c")

    @pl.when(wid == 0)
    def _():
        pltpu.sync_copy(mask_hbm, mask_v)
        pltpu.sync_copy(steps_hbm, steps_v)
        mask = mask_v[...]
        stp = steps_v[...]
        unm = mask == 0
        unm_i = jnp.where(unm, jnp.int32(1), jnp.int32(0))
        steps_o_v[...] = jnp.where(unm, stp, jnp.int32(0))

        iota = lax.iota(jnp.int32, 16)
        key = unm_i * 16 + iota
        _, perm = lax.sort((key, iota), num_keys=1)
        perm_v[...] = perm
        unm_v[...] = unm_i
        unm_p = plsc.load_gather(unm_v, [perm])
        maskp_v[...] = 1 - unm_p

        any_u = plsc.all_reduce_population_count(unm_p == 1)
        ffs = plsc.all_reduce_ffs(unm_p == 1)
        safe_ffs = jnp.where(any_u > 0, ffs, jnp.int32(0))
        first_row = plsc.load_gather(perm_v, [safe_ffs])
        src_v[...] = jnp.where(unm_p == 1, perm, first_row)

        pltpu.sync_copy(steps_o_v, steps_out_hbm)
        pltpu.sync_copy(maskp_v, maskp_hbm)
        pltpu.sync_copy(perm_v, perm_hbm)
        pltpu.sync_copy(src_v, src_hbm)


def _sc_routing(mask_i, steps):
    import functools
    kern = functools.partial(
        pl.kernel,
        out_type=[jax.ShapeDtypeStruct((_B,), jnp.int32)] * 4,
        mesh=plsc.VectorSubcoreMesh(core_axis_name="c", subcore_axis_name="s"),
        scratch_types=[pltpu.VMEM((_B,), jnp.int32)] * 7,
        compiler_params=pltpu.CompilerParams(needs_layout_passes=False),
    )(_routing_body)
    return kern(mask_i, steps)


def _rows_body(maskp_ref, perm_ref, src_ref, pred_ref, z_ref,
               pi_ref, zi_ref, po_ref, zo_ref):
    t = pl.program_id(1)
    m = maskp_ref[t] != 0

    @pl.when(jnp.logical_and(m, t < 2))
    def _():
        po_ref[0] = jnp.broadcast_to(pi_ref[0], po_ref.shape[1:])
        zo_ref[0] = jnp.broadcast_to(zi_ref[0], zo_ref.shape[1:])

    @pl.when(jnp.logical_not(m))
    def _():
        po_ref[...] = pred_ref[...]
        zo_ref[...] = z_ref[...]


def kernel(prediction_y, reasoning_Z, steps, mask, pred_init, Z_init):
    B, L, D = prediction_y.shape
    J = L // _LB
    mask_i = mask.astype(jnp.int32)
    steps_out, mask_p, perm, src_row = _sc_routing(mask_i, steps)

    def in_map(j, t, maskp_ref, perm_ref, src_ref):
        return (src_ref[t], j, 0)

    def out_map(j, t, maskp_ref, perm_ref, src_ref):
        return (perm_ref[t], j, 0)

    def init_map(j, t, maskp_ref, perm_ref, src_ref):
        return (0, 0, 0)

    grid_spec = pltpu.PrefetchScalarGridSpec(
        num_scalar_prefetch=3,
        grid=(J, B),
        in_specs=[
            pl.BlockSpec((1, _LB, D), in_map),
            pl.BlockSpec((1, _LB, D), in_map),
            pl.BlockSpec((1, 1, D), init_map),
            pl.BlockSpec((1, 1, D), init_map),
        ],
        out_specs=[
            pl.BlockSpec((1, _LB, D), out_map),
            pl.BlockSpec((1, _LB, D), out_map),
        ],
    )
    pred_out, Z_out = pl.pallas_call(
        _rows_body,
        grid_spec=grid_spec,
        out_shape=[jax.ShapeDtypeStruct((B, L, D), jnp.float32)] * 2,
    )(mask_p, perm, src_row, prediction_y, reasoning_Z, pred_init, Z_init)
    return (pred_out, Z_out, steps_out)
